# baseline (speedup 1.0000x reference)
import functools

import jax
import jax.numpy as jnp
import numpy as np
from jax import lax
from jax.experimental import pallas as pl
from jax.experimental.pallas import tpu as pltpu
from jax.experimental.pallas import tpu_sc as plsc

_S = 10000
_EX = 10000
_K = 128
_D = 32
_B = 16384

_INFO = plsc.get_sparse_core_info()
_NW = _INFO.num_cores * _INFO.num_subcores
_BPW = _B // _NW


_CH = 128
_NCH = _BPW // _CH
_L = 16


def _sc_gather(stu_id_h, ex_id_h, sp_h, ep_h, ed_h,
               out_s_h, out_e_h, out_d_h,
               sidx, eidx, qs, qe, sb, kb, cs, ck, edv, sem, seme):
    wid = lax.axis_index("s") * _INFO.num_cores + lax.axis_index("c")
    base = wid * _BPW
    pltpu.sync_copy(stu_id_h.at[pl.ds(base, _BPW)], sidx)
    pltpu.sync_copy(ex_id_h.at[pl.ds(base, _BPW)], eidx)
    ce = pltpu.async_copy(ed_h.at[eidx], edv, seme)
    for i in range(_BPW // _L):
        sl = pl.ds(i * _L, _L)
        qs[sl] = lax.shift_right_logical(sidx[sl], 2)
        qe[sl] = lax.shift_right_logical(eidx[sl], 2)
    for c in range(_NCH):
        sl = pl.ds(c * _CH, _CH)
        c1 = pltpu.async_copy(sp_h.at[qs.at[sl]], sb, sem)
        c2 = pltpu.async_copy(ep_h.at[qe.at[sl]], kb, sem)
        c1.wait()
        c2.wait()

        def grp_sel(gi, _):
            r0 = gi * _L
            g0 = c * _CH + r0
            siv = (sidx[pl.ds(g0, _L)] & 3) * _D
            eiv = (eidx[pl.ds(g0, _L)] & 3) * _D
            for j in range(_L):
                soff = siv[j]
                eoff = eiv[j]
                o = (g0 + j) * _D
                cs[pl.ds(o, _L)] = sb[r0 + j, pl.ds(soff, _L)]
                cs[pl.ds(o + _L, _L)] = sb[r0 + j, pl.ds(soff + _L, _L)]
                ck[pl.ds(o, _L)] = kb[r0 + j, pl.ds(eoff, _L)]
                ck[pl.ds(o + _L, _L)] = kb[r0 + j, pl.ds(eoff + _L, _L)]
            return _

        lax.fori_loop(0, _CH // _L, grp_sel, 0, unroll=False)
    pltpu.sync_copy(cs, out_s_h.at[pl.ds(base * _D, _BPW * _D)])
    pltpu.sync_copy(ck, out_e_h.at[pl.ds(base * _D, _BPW * _D)])
    ce.wait()
    pltpu.sync_copy(edv, out_d_h.at[pl.ds(base, _BPW)])


_sc_gather_call = functools.partial(
    pl.kernel,
    mesh=plsc.VectorSubcoreMesh(core_axis_name="c", subcore_axis_name="s"),
    out_type=[
        jax.ShapeDtypeStruct((_B * _D,), jnp.float32),
        jax.ShapeDtypeStruct((_B * _D,), jnp.float32),
        jax.ShapeDtypeStruct((_B,), jnp.float32),
    ],
    scratch_types=[
        pltpu.VMEM((_BPW,), jnp.int32),
        pltpu.VMEM((_BPW,), jnp.int32),
        pltpu.VMEM((_BPW,), jnp.int32),
        pltpu.VMEM((_BPW,), jnp.int32),
        pltpu.VMEM((_CH, _K), jnp.float32),
        pltpu.VMEM((_CH, _K), jnp.float32),
        pltpu.VMEM((_BPW * _D,), jnp.float32),
        pltpu.VMEM((_BPW * _D,), jnp.float32),
        pltpu.VMEM((_BPW,), jnp.float32),
        pltpu.SemaphoreType.DMA,
        pltpu.SemaphoreType.DMA,
    ],
)(_sc_gather)


def _tc_main(gs_ref, ge_ref, kemb_ref, ikp_ref, ed_ref, ones_ref, out_ref):
    dn = (((1,), (1,)), ((), ()))
    kemb = kemb_ref[...]
    se = jax.nn.sigmoid(
        lax.dot_general(gs_ref[...], kemb, dn,
                        preferred_element_type=jnp.float32))
    kd = lax.dot_general(ge_ref[...], kemb, dn,
                         preferred_element_type=jnp.float32)
    ikp = ikp_ref[...]
    prod = ikp * (se - kd)
    ones = ones_ref[...]
    dno = (((1,), (1,)), ((), ()))
    num = lax.dot_general(ones, prod, dno, preferred_element_type=jnp.float32)
    den = lax.dot_general(ones, ikp, dno, preferred_element_type=jnp.float32)
    ed = jax.nn.sigmoid(ed_ref[0])
    out_ref[0] = jax.nn.sigmoid(ed * num / den)


def kernel(stu_id, input_exercise, input_knowledge_point, student_emb,
           exercise_emb, knowledge_emb, e_disc, edge_index_1, edge_vals_1,
           edge_index_0, edge_vals_0, d_i_1, d_j_1, d_i_0, d_j_0):
    sid = stu_id.astype(jnp.int32)
    eid = input_exercise.astype(jnp.int32)
    gstu, gex, ged = _sc_gather_call(sid, eid,
                                     student_emb.reshape(_S // 4, _K),
                                     exercise_emb.reshape(_EX // 4, _K),
                                     e_disc.reshape(-1))

    bb = 2048
    grid = _B // bb
    ones = np.ones((1, _K), np.float32)
    out = pl.pallas_call(
        _tc_main,
        grid=(grid,),
        in_specs=[
            pl.BlockSpec((bb, _D), lambda i: (i, 0)),
            pl.BlockSpec((bb, _D), lambda i: (i, 0)),
            pl.BlockSpec((_K, _D), lambda i: (0, 0)),
            pl.BlockSpec((bb, _K), lambda i: (i, 0)),
            pl.BlockSpec((1, 1, bb), lambda i: (i, 0, 0)),
            pl.BlockSpec((1, _K), lambda i: (0, 0)),
        ],
        out_specs=pl.BlockSpec((1, 1, bb), lambda i: (i, 0, 0)),
        out_shape=jax.ShapeDtypeStruct((grid, 1, bb), jnp.float32),
    )(gstu.reshape(_B, _D), gex.reshape(_B, _D), knowledge_emb,
      input_knowledge_point, ged.reshape(grid, 1, bb), ones)
    return out.reshape(-1)


# speedup vs baseline: 1.1495x; 1.1495x over previous
import functools

import jax
import jax.numpy as jnp
import numpy as np
from jax import lax
from jax.experimental import pallas as pl
from jax.experimental.pallas import tpu as pltpu
from jax.experimental.pallas import tpu_sc as plsc

_S = 10000
_EX = 10000
_K = 128
_D = 32
_B = 16384

_INFO = plsc.get_sparse_core_info()
_NW = _INFO.num_cores * _INFO.num_subcores
_BPW = _B // _NW
_CH = 128
_NCH = _BPW // _CH
_L = 16


def _tc_tables(st_ref, ex_ref, kemb_ref, stat_ref, kd_ref):
    dn = (((1,), (1,)), ((), ()))
    kemb = kemb_ref[...]
    stat_ref[...] = jax.nn.sigmoid(
        lax.dot_general(st_ref[...], kemb, dn,
                        preferred_element_type=jnp.float32))
    kd_ref[...] = lax.dot_general(ex_ref[...], kemb, dn,
                                  preferred_element_type=jnp.float32)


def _sc_gather(stu_id_h, ex_id_h, stat_h, kd_h, ed_h, ikp_h,
               num_h, den_h, out_e_h,
               sidx, eidx, sb0, kb0, pb0, sb1, kb1, pb1, nb, db, edv,
               sem0, sem1, seme):
    wid = lax.axis_index("s") * _INFO.num_cores + lax.axis_index("c")
    base = wid * _BPW
    pltpu.sync_copy(stu_id_h.at[pl.ds(base, _BPW)], sidx)
    pltpu.sync_copy(ex_id_h.at[pl.ds(base, _BPW)], eidx)
    ce = pltpu.async_copy(ed_h.at[eidx], edv, seme)
    sbufs = (sb0, sb1)
    kbufs = (kb0, kb1)
    pbufs = (pb0, pb1)
    sems = (sem0, sem1)

    def issue(c):
        sl = pl.ds(c * _CH, _CH)
        return (pltpu.async_copy(stat_h.at[sidx.at[sl]], sbufs[c % 2],
                                 sems[c % 2]),
                pltpu.async_copy(kd_h.at[eidx.at[sl]], kbufs[c % 2],
                                 sems[c % 2]),
                pltpu.async_copy(ikp_h.at[pl.ds(base + c * _CH, _CH)],
                                 pbufs[c % 2], sems[c % 2]))

    cps = issue(0)
    for c in range(_NCH):
        nxt = issue(c + 1) if c + 1 < _NCH else None
        for cp in cps:
            cp.wait()
        sb, kb, pb = sbufs[c % 2], kbufs[c % 2], pbufs[c % 2]

        def row_red(r, _):
            s = sb[r, pl.ds(0, _L)]
            k = kb[r, pl.ds(0, _L)]
            p = pb[r, pl.ds(0, _L)]
            an = p * (s - k)
            ad = p
            for c8 in range(1, _K // _L):
                sl = pl.ds(c8 * _L, _L)
                s = sb[r, sl]
                k = kb[r, sl]
                p = pb[r, sl]
                an = an + p * (s - k)
                ad = ad + p
            nb[pl.ds(r * _L, _L)] = an
            db[pl.ds(r * _L, _L)] = ad
            return _

        lax.fori_loop(0, _CH, row_red, 0, unroll=False)
        osl = pl.ds((base + c * _CH) * _L, _CH * _L)
        pltpu.sync_copy(nb, num_h.at[osl])
        pltpu.sync_copy(db, den_h.at[osl])
        cps = nxt
    ce.wait()
    pltpu.sync_copy(edv, out_e_h.at[pl.ds(base, _BPW)])


_sc_gather_call = functools.partial(
    pl.kernel,
    mesh=plsc.VectorSubcoreMesh(core_axis_name="c", subcore_axis_name="s"),
    out_type=[
        jax.ShapeDtypeStruct((_B * _L,), jnp.float32),
        jax.ShapeDtypeStruct((_B * _L,), jnp.float32),
        jax.ShapeDtypeStruct((_B,), jnp.float32),
    ],
    scratch_types=[
        pltpu.VMEM((_BPW,), jnp.int32),
        pltpu.VMEM((_BPW,), jnp.int32),
        pltpu.VMEM((_CH, _K), jnp.float32),
        pltpu.VMEM((_CH, _K), jnp.float32),
        pltpu.VMEM((_CH, _K), jnp.float32),
        pltpu.VMEM((_CH, _K), jnp.float32),
        pltpu.VMEM((_CH, _K), jnp.float32),
        pltpu.VMEM((_CH, _K), jnp.float32),
        pltpu.VMEM((_CH * _L,), jnp.float32),
        pltpu.VMEM((_CH * _L,), jnp.float32),
        pltpu.VMEM((_BPW,), jnp.float32),
        pltpu.SemaphoreType.DMA,
        pltpu.SemaphoreType.DMA,
        pltpu.SemaphoreType.DMA,
    ],
)(_sc_gather)


def _tc_tail(n_ref, d_ref, ed_ref, g_ref, out_ref):
    dn = (((1,), (0,)), ((), ()))
    g = g_ref[...]
    num = lax.dot_general(n_ref[...], g, dn,
                          preferred_element_type=jnp.float32)
    den = lax.dot_general(d_ref[...], g, dn,
                          preferred_element_type=jnp.float32)
    ed = jax.nn.sigmoid(ed_ref[...])
    out_ref[...] = jax.nn.sigmoid(ed * num / den)


def kernel(stu_id, input_exercise, input_knowledge_point, student_emb,
           exercise_emb, knowledge_emb, e_disc, edge_index_1, edge_vals_1,
           edge_index_0, edge_vals_0, d_i_1, d_j_1, d_i_0, d_j_0):
    rb = 2000
    stat_tab, kd_tab = pl.pallas_call(
        _tc_tables,
        grid=(_S // rb,),
        in_specs=[
            pl.BlockSpec((rb, _D), lambda i: (i, 0)),
            pl.BlockSpec((rb, _D), lambda i: (i, 0)),
            pl.BlockSpec((_K, _D), lambda i: (0, 0)),
        ],
        out_specs=[
            pl.BlockSpec((rb, _K), lambda i: (i, 0)),
            pl.BlockSpec((rb, _K), lambda i: (i, 0)),
        ],
        out_shape=[
            jax.ShapeDtypeStruct((_S, _K), jnp.float32),
            jax.ShapeDtypeStruct((_EX, _K), jnp.float32),
        ],
    )(student_emb, exercise_emb, knowledge_emb)

    sid = stu_id.astype(jnp.int32)
    eid = input_exercise.astype(jnp.int32)
    num16, den16, ged = _sc_gather_call(sid, eid, stat_tab, kd_tab,
                                        e_disc.reshape(-1),
                                        input_knowledge_point)

    rows = _B * _L // _K
    gmat = np.zeros((_K, 8), np.float32)
    for g in range(8):
        gmat[g * _L:(g + 1) * _L, g] = 1.0
    out = pl.pallas_call(
        _tc_tail,
        in_specs=[
            pl.BlockSpec((rows, _K), lambda: (0, 0)),
            pl.BlockSpec((rows, _K), lambda: (0, 0)),
            pl.BlockSpec((rows, 8), lambda: (0, 0)),
            pl.BlockSpec((_K, 8), lambda: (0, 0)),
        ],
        out_specs=pl.BlockSpec((rows, 8), lambda: (0, 0)),
        out_shape=jax.ShapeDtypeStruct((rows, 8), jnp.float32),
    )(num16.reshape(rows, _K), den16.reshape(rows, _K),
      ged.reshape(rows, 8), gmat)
    return out.reshape(-1)
